# baseline (speedup 1.0000x reference)
import math

import jax
import jax.numpy as jnp
from jax import lax
from jax.experimental import pallas as pl
from jax.experimental.pallas import tpu as pltpu
from jax.experimental.pallas import tpu_sc as plsc

_IMG = 1024.0
_N_LEVELS = 16
_LOG2T = 19
_MASK = (1 << _LOG2T) - 1
_PRIME = 2654435761
_B = 1048576

_NC, _NS = 2, 16
_NW = _NC * _NS
_C = 256
_PW = _B // _NW
_NCHUNK = _PW // _C

_N_RESIDENT = 8
_N_STREAM = _N_LEVELS - _N_RESIDENT


def _level_res():
    b = math.exp((math.log(2048) - math.log(16)) / (_N_LEVELS - 1))
    return [math.floor(16 * (b ** i)) for i in range(_N_LEVELS)]


_RES = _level_res()
_DENSE = [r * r < (1 << _LOG2T) for r in _RES]
_ROWS = [(r + 1) ** 2 if d else (1 << _LOG2T) for r, d in zip(_RES, _DENSE)]

_SPMEM_LEVELS = (8, 9, 10)
_HBM_LEVELS = (11, 12, 13, 14, 15)
_SH_OFF = {}
_off = 0
for _l in _SPMEM_LEVELS:
    _SH_OFF[_l] = _off
    _off += -(-_ROWS[_l] // 16) * 16
_SH_WORDS = _off


def _sc_body(x_hbm, *rest):
    nh = len(_HBM_LEVELS)
    tabs_hbm = (rest[:_N_RESIDENT] + (None,) * len(_SPMEM_LEVELS)
                + rest[_N_RESIDENT + 1:_N_RESIDENT + 1 + nh])
    mid_hbm = rest[_N_RESIDENT]
    out_hbm = rest[_N_RESIDENT + 1 + nh]
    r = _N_RESIDENT + 2 + nh
    xy_v, xt_v = rest[r], rest[r + 1]
    idx_v = rest[r + 2:r + 2 + _N_STREAM]
    rows_v = rest[r + 2 + _N_STREAM:r + 2 + 2 * _N_STREAM]
    w_v = rest[r + 2 + 2 * _N_STREAM:r + 2 + 3 * _N_STREAM]
    k = r + 2 + 3 * _N_STREAM
    out_v = rest[k]
    tv = rest[k + 1:k + 1 + _N_RESIDENT]
    sh_v = rest[k + 1 + _N_RESIDENT]
    sems = rest[k + 2 + _N_RESIDENT]
    hsems = rest[k + 3 + _N_RESIDENT]
    outsem = rest[k + 4 + _N_RESIDENT]

    sid = lax.axis_index("s")
    wid = sid * _NC + lax.axis_index("c")
    ii = lax.iota(jnp.int32, 16)
    zz = jnp.zeros((16,), jnp.int32)

    for l in range(_N_RESIDENT):
        pltpu.sync_copy(tabs_hbm[l], tv[l])

    @pl.when(sid == 0)
    def _():
        pltpu.sync_copy(mid_hbm, sh_v)

    plsc.subcore_barrier()

    def point_setup(p):
        x0 = xt_v[pl.ds(p, 16)]
        x1 = xt_v[pl.ds(_C + p, 16)]
        return x0, x1

    def floors(x0, x1, res):
        inv = 1.0 / (_IMG / res)
        t0 = x0 * inv
        t1 = x1 * inv
        bl0 = t0.astype(jnp.int32)
        bl1 = t1.astype(jnp.int32)
        w0 = t0 - bl0.astype(jnp.float32)
        w1 = t1 - bl1.astype(jnp.float32)
        return bl0, bl1, w0, w1

    def corner_idx(bl0, bl1, l):
        res = _RES[l]
        if _DENSE[l]:
            b = bl0 * res + bl1 + _SH_OFF.get(l, 0)
            return b, b + 1, b + res, b + res + 1
        u0 = bl0.astype(jnp.uint32)
        u1b = bl1.astype(jnp.uint32) * jnp.uint32(_PRIME)
        u1b1 = u1b + jnp.uint32(_PRIME)
        u0p = u0 + jnp.uint32(1)
        m = jnp.uint32(_MASK)
        i00 = ((u0 ^ u1b) & m).astype(jnp.int32)
        i01 = ((u0 ^ u1b1) & m).astype(jnp.int32)
        i10 = ((u0p ^ u1b) & m).astype(jnp.int32)
        i11 = ((u0p ^ u1b1) & m).astype(jnp.int32)
        return i00, i01, i10, i11

    def unpack(w):
        e0 = plsc.bitcast(w << 16, jnp.float32)
        e1 = plsc.bitcast(w & jnp.int32(-65536), jnp.float32)
        return e0, e1

    def combine_store(e, w0, w1, rr32, l):
        for f in range(2):
            q0 = e[0][f] + (e[1][f] - e[0][f]) * w1
            q1 = e[2][f] + (e[3][f] - e[2][f]) * w1
            o = q0 + (q1 - q0) * w0
            plsc.store_scatter(out_v, [rr32 + (2 * l + f)], o)

    def idx_pass():
        def body(g, c):
            p = g * 16
            x0, x1 = point_setup(p)
            rp4 = (p + ii) * 4
            for j in range(_N_STREAM):
                l = _N_RESIDENT + j
                bl0, bl1, w0, w1 = floors(x0, x1, _RES[l])
                w_v[j][pl.ds(p, 16)] = w0
                w_v[j][pl.ds(_C + p, 16)] = w1
                iks = corner_idx(bl0, bl1, l)
                for q in range(4):
                    plsc.store_scatter(idx_v[j], [rp4 + q], iks[q])
            return c

        lax.fori_loop(0, _C // 16, body, 0)

    def fire(j):
        l = _N_RESIDENT + j
        src = sh_v if l in _SPMEM_LEVELS else tabs_hbm[l]
        if l in _SPMEM_LEVELS:
            return (pltpu.async_copy(src.at[idx_v[j]], rows_v[j], sems[j]),)
        h = 2 * _C
        return (
            pltpu.async_copy(
                src.at[idx_v[j].at[pl.ds(0, h)]],
                rows_v[j].at[pl.ds(0, h)], sems[j]),
            pltpu.async_copy(
                src.at[idx_v[j].at[pl.ds(h, h)]],
                rows_v[j].at[pl.ds(h, h)], hsems[j]),
        )

    def comb_pass(j):
        l = _N_RESIDENT + j
        rv, wv = rows_v[j], w_v[j]

        def body(g, c):
            p = g * 16
            w0 = wv[pl.ds(p, 16)]
            w1 = wv[pl.ds(_C + p, 16)]
            rp4 = (p + ii) * 4
            e = tuple(
                unpack(plsc.load_gather(rv, [rp4 + q])) for q in range(4))
            combine_store(e, w0, w1, (p + ii) * 32, l)
            return c

        lax.fori_loop(0, _C // 16, body, 0)

    def out_wait():
        pltpu.make_async_copy(
            out_v, out_hbm.at[pl.ds(0, 32 * _C)], outsem).wait()

    def chunk_body(ci, carry):
        base = wid * _PW + ci * _C
        pltpu.sync_copy(x_hbm.at[pl.ds(2 * base, 2 * _C)], xy_v)

        def tr_body(g, c):
            p = g * 16
            r2 = (p + ii) * 2
            xt_v[pl.ds(p, 16)] = plsc.load_gather(xy_v, [r2])
            xt_v[pl.ds(_C + p, 16)] = plsc.load_gather(xy_v, [r2 + 1])
            return c

        lax.fori_loop(0, _C // 16, tr_body, 0)

        idx_pass()
        dscs = [fire(j) for j in range(_N_STREAM)]

        @pl.when(ci > 0)
        def _():
            out_wait()

        def res_body(g, c):
            p = g * 16
            x0, x1 = point_setup(p)
            rr32 = (p + ii) * 32
            for l in range(_N_RESIDENT):
                bl0, bl1, w0, w1 = floors(x0, x1, _RES[l])
                iks = corner_idx(bl0, bl1, l)
                e = tuple(
                    unpack(plsc.load_gather(tv[l], [ik])) for ik in iks)
                combine_store(e, w0, w1, rr32, l)
            return c

        lax.fori_loop(0, _C // 16, res_body, 0)

        for j in range(_N_STREAM):
            for d in dscs[j]:
                d.wait()
            comb_pass(j)

        pltpu.async_copy(out_v, out_hbm.at[pl.ds(32 * base, 32 * _C)], outsem)
        return carry

    lax.fori_loop(0, _NCHUNK, chunk_body, 0)
    out_wait()


def kernel(x, tables):
    mesh = plsc.VectorSubcoreMesh(core_axis_name="c", subcore_axis_name="s")
    scratch = [
        pltpu.VMEM((2 * _C,), jnp.float32),
        pltpu.VMEM((2 * _C,), jnp.float32),
    ] + [
        pltpu.VMEM((4 * _C,), jnp.int32) for _ in range(_N_STREAM)
    ] + [
        pltpu.VMEM((4 * _C,), jnp.int32) for _ in range(_N_STREAM)
    ] + [
        pltpu.VMEM((2 * _C,), jnp.float32) for _ in range(_N_STREAM)
    ] + [
        pltpu.VMEM((32 * _C,), jnp.float32),
    ] + [
        pltpu.VMEM((_ROWS[l],), jnp.int32) for l in range(_N_RESIDENT)
    ] + [
        pltpu.VMEM_SHARED((_SH_WORDS,), jnp.int32),
        tuple(pltpu.SemaphoreType.DMA for _ in range(_N_STREAM)),
        tuple(pltpu.SemaphoreType.DMA for _ in range(_N_STREAM)),
        pltpu.SemaphoreType.DMA,
    ]
    fn = pl.kernel(
        _sc_body,
        out_type=jax.ShapeDtypeStruct((_B * 32,), jnp.float32),
        mesh=mesh,
        scratch_types=scratch,
        compiler_params=pltpu.CompilerParams(needs_layout_passes=False),
        name="ngp_sc",
    )

    def _pack(t):
        b = t.astype(jnp.bfloat16)
        u = jax.lax.bitcast_convert_type(b, jnp.uint16).astype(jnp.uint32)
        w = u[:, 0] | (u[:, 1] << 16)
        return jax.lax.bitcast_convert_type(w, jnp.int32)

    mid_parts = []
    for l in _SPMEM_LEVELS:
        f = _pack(tables[l])
        pad = -(-_ROWS[l] // 16) * 16 - _ROWS[l]
        mid_parts.append(f)
        if pad:
            mid_parts.append(jnp.zeros((pad,), jnp.int32))
    out = fn(
        x.reshape(-1),
        *(_pack(tables[l]) for l in range(_N_RESIDENT)),
        jnp.concatenate(mid_parts),
        *(_pack(tables[l]) for l in _HBM_LEVELS),
    )
    return out.reshape(_B, 32)


# speedup vs baseline: 1.1227x; 1.1227x over previous
import math

import jax
import jax.numpy as jnp
from jax import lax
from jax.experimental import pallas as pl
from jax.experimental.pallas import tpu as pltpu
from jax.experimental.pallas import tpu_sc as plsc

_IMG = 1024.0
_N_LEVELS = 16
_LOG2T = 19
_MASK = (1 << _LOG2T) - 1
_PRIME = 2654435761
_B = 1048576

_NC, _NS = 2, 16
_NW = _NC * _NS
_C = 256
_PW = _B // _NW
_NCHUNK = _PW // _C

_N_RESIDENT = 8
_N_STREAM = _N_LEVELS - _N_RESIDENT


def _level_res():
    b = math.exp((math.log(2048) - math.log(16)) / (_N_LEVELS - 1))
    return [math.floor(16 * (b ** i)) for i in range(_N_LEVELS)]


_RES = _level_res()
_DENSE = [r * r < (1 << _LOG2T) for r in _RES]
_ROWS = [(r + 1) ** 2 if d else (1 << _LOG2T) for r, d in zip(_RES, _DENSE)]

_SPMEM_LEVELS = (8, 9, 10)
_HBM_LEVELS = (11, 12, 13, 14, 15)
_SH_OFF = {}
_off = 0
for _l in _SPMEM_LEVELS:
    _SH_OFF[_l] = _off
    _off += -(-_ROWS[_l] // 16) * 16
_SH_WORDS = _off


def _sc_body(x_hbm, *rest):
    nh = len(_HBM_LEVELS)
    tabs_hbm = (rest[:_N_RESIDENT] + (None,) * len(_SPMEM_LEVELS)
                + rest[_N_RESIDENT + 1:_N_RESIDENT + 1 + nh])
    mid_hbm = rest[_N_RESIDENT]
    out_hbm = rest[_N_RESIDENT + 1 + nh]
    r = _N_RESIDENT + 2 + nh
    xy_v, xt_v = rest[r], rest[r + 1]
    idx_v = rest[r + 2:r + 2 + _N_STREAM]
    rows_v = rest[r + 2 + _N_STREAM:r + 2 + 2 * _N_STREAM]
    w_v = rest[r + 2 + 2 * _N_STREAM:r + 2 + 3 * _N_STREAM]
    k = r + 2 + 3 * _N_STREAM
    out_v = rest[k]
    tv = rest[k + 1:k + 1 + _N_RESIDENT]
    sh_v = rest[k + 1 + _N_RESIDENT]
    sems = rest[k + 2 + _N_RESIDENT]
    outsem = rest[k + 3 + _N_RESIDENT]

    sid = lax.axis_index("s")
    wid = sid * _NC + lax.axis_index("c")
    ii = lax.iota(jnp.int32, 16)
    zz = jnp.zeros((16,), jnp.int32)

    for l in range(_N_RESIDENT):
        pltpu.sync_copy(tabs_hbm[l], tv[l])

    @pl.when(sid == 0)
    def _():
        pltpu.sync_copy(mid_hbm, sh_v)

    plsc.subcore_barrier()

    def point_setup(p):
        x0 = xt_v[pl.ds(p, 16)]
        x1 = xt_v[pl.ds(_C + p, 16)]
        return x0, x1

    def floors(x0, x1, res):
        inv = 1.0 / (_IMG / res)
        t0 = x0 * inv
        t1 = x1 * inv
        bl0 = t0.astype(jnp.int32)
        bl1 = t1.astype(jnp.int32)
        w0 = t0 - bl0.astype(jnp.float32)
        w1 = t1 - bl1.astype(jnp.float32)
        return bl0, bl1, w0, w1

    def corner_idx(bl0, bl1, l):
        res = _RES[l]
        if _DENSE[l]:
            b = bl0 * res + bl1 + _SH_OFF.get(l, 0)
            return b, b + 1, b + res, b + res + 1
        u0 = bl0.astype(jnp.uint32)
        u1b = bl1.astype(jnp.uint32) * jnp.uint32(_PRIME)
        u1b1 = u1b + jnp.uint32(_PRIME)
        u0p = u0 + jnp.uint32(1)
        m = jnp.uint32(_MASK)
        i00 = ((u0 ^ u1b) & m).astype(jnp.int32)
        i01 = ((u0 ^ u1b1) & m).astype(jnp.int32)
        i10 = ((u0p ^ u1b) & m).astype(jnp.int32)
        i11 = ((u0p ^ u1b1) & m).astype(jnp.int32)
        return i00, i01, i10, i11

    def unpack(w):
        e0 = plsc.bitcast(w << 16, jnp.float32)
        e1 = plsc.bitcast(w & jnp.int32(-65536), jnp.float32)
        return e0, e1

    def combine_store(e, w0, w1, rr, l):
        for f in range(2):
            q0 = e[0][f] + (e[1][f] - e[0][f]) * w1
            q1 = e[2][f] + (e[3][f] - e[2][f]) * w1
            o = q0 + (q1 - q0) * w0
            plsc.store_scatter(out_v, [rr, zz + (2 * l + f)], o)

    def idx_pass():
        def body(g, c):
            p = g * 16
            x0, x1 = point_setup(p)
            rp4 = (p + ii) * 4
            for j in range(_N_STREAM):
                l = _N_RESIDENT + j
                bl0, bl1, w0, w1 = floors(x0, x1, _RES[l])
                w_v[j][pl.ds(p, 16)] = w0
                w_v[j][pl.ds(_C + p, 16)] = w1
                iks = corner_idx(bl0, bl1, l)
                for q in range(4):
                    plsc.store_scatter(idx_v[j], [rp4 + q], iks[q])
            return c

        lax.fori_loop(0, _C // 16, body, 0)

    def fire(j):
        l = _N_RESIDENT + j
        src = sh_v if l in _SPMEM_LEVELS else tabs_hbm[l]
        return pltpu.async_copy(src.at[idx_v[j]], rows_v[j], sems[j])

    def comb_pass(j):
        l = _N_RESIDENT + j
        rv, wv = rows_v[j], w_v[j]

        def body(g, c):
            p = g * 16
            w0 = wv[pl.ds(p, 16)]
            w1 = wv[pl.ds(_C + p, 16)]
            rp4 = (p + ii) * 4
            e = tuple(
                unpack(plsc.load_gather(rv, [rp4 + q])) for q in range(4))
            combine_store(e, w0, w1, p + ii, l)
            return c

        lax.fori_loop(0, _C // 16, body, 0)

    def out_wait():
        pltpu.make_async_copy(
            out_v, out_hbm.at[pl.ds(0, _C)], outsem).wait()

    def chunk_body(ci, carry):
        base = wid * _PW + ci * _C
        pltpu.sync_copy(x_hbm.at[pl.ds(2 * base, 2 * _C)], xy_v)

        def tr_body(g, c):
            p = g * 16
            r2 = (p + ii) * 2
            xt_v[pl.ds(p, 16)] = plsc.load_gather(xy_v, [r2])
            xt_v[pl.ds(_C + p, 16)] = plsc.load_gather(xy_v, [r2 + 1])
            return c

        lax.fori_loop(0, _C // 16, tr_body, 0)

        idx_pass()
        dscs = [fire(j) for j in range(_N_STREAM)]

        @pl.when(ci > 0)
        def _():
            out_wait()

        def res_body(g, c):
            p = g * 16
            x0, x1 = point_setup(p)
            rr = p + ii
            for l in range(_N_RESIDENT):
                bl0, bl1, w0, w1 = floors(x0, x1, _RES[l])
                iks = corner_idx(bl0, bl1, l)
                e = tuple(
                    unpack(plsc.load_gather(tv[l], [ik])) for ik in iks)
                combine_store(e, w0, w1, rr, l)
            return c

        lax.fori_loop(0, _C // 16, res_body, 0)

        for j in range(_N_STREAM):
            dscs[j].wait()
            comb_pass(j)

        pltpu.async_copy(out_v, out_hbm.at[pl.ds(base, _C)], outsem)
        return carry

    lax.fori_loop(0, _NCHUNK, chunk_body, 0)
    out_wait()


def kernel(x, tables):
    mesh = plsc.VectorSubcoreMesh(core_axis_name="c", subcore_axis_name="s")
    scratch = [
        pltpu.VMEM((2 * _C,), jnp.float32),
        pltpu.VMEM((2 * _C,), jnp.float32),
    ] + [
        pltpu.VMEM((4 * _C,), jnp.int32) for _ in range(_N_STREAM)
    ] + [
        pltpu.VMEM((4 * _C,), jnp.int32) for _ in range(_N_STREAM)
    ] + [
        pltpu.VMEM((2 * _C,), jnp.float32) for _ in range(_N_STREAM)
    ] + [
        pltpu.VMEM((_C, 128), jnp.float32),
    ] + [
        pltpu.VMEM((_ROWS[l],), jnp.int32) for l in range(_N_RESIDENT)
    ] + [
        pltpu.VMEM_SHARED((_SH_WORDS,), jnp.int32),
        tuple(pltpu.SemaphoreType.DMA for _ in range(_N_STREAM)),
        pltpu.SemaphoreType.DMA,
    ]
    fn = pl.kernel(
        _sc_body,
        out_type=jax.ShapeDtypeStruct((_B, 128), jnp.float32),
        mesh=mesh,
        scratch_types=scratch,
        compiler_params=pltpu.CompilerParams(needs_layout_passes=False),
        name="ngp_sc",
    )

    def _pack(t):
        b = t.astype(jnp.bfloat16)
        u = jax.lax.bitcast_convert_type(b, jnp.uint16).astype(jnp.uint32)
        w = u[:, 0] | (u[:, 1] << 16)
        return jax.lax.bitcast_convert_type(w, jnp.int32)

    mid_parts = []
    for l in _SPMEM_LEVELS:
        f = _pack(tables[l])
        pad = -(-_ROWS[l] // 16) * 16 - _ROWS[l]
        mid_parts.append(f)
        if pad:
            mid_parts.append(jnp.zeros((pad,), jnp.int32))
    out = fn(
        x.reshape(-1),
        *(_pack(tables[l]) for l in range(_N_RESIDENT)),
        jnp.concatenate(mid_parts),
        *(_pack(tables[l]) for l in _HBM_LEVELS),
    )
    return out[:, :32]
